# baseline (speedup 1.0000x reference)
import functools

import jax
import jax.numpy as jnp
from jax import lax
from jax.experimental import pallas as pl
from jax.experimental.pallas import tpu as pltpu
from jax.experimental.pallas import tpu_sc as plsc

B = 4096
H = 2048
E = 8
THRESH = 0.1

BLK_B = 512
BLK_M = 512
NBLK = 24
R = NBLK * BLK_M
NBLK_G = NBLK + 1
ZROW = R
SENT = 1 << 20

NC, NS = 2, 16
NW = NC * NS
RPW = R // NW
GCH = 24
DCH = 4096
TPW = B // NW
CCH = 8


def _gate_body(x_ref, gW1_ref, gb1_ref, gW2_ref, gb2_ref, gw_ref, w_ref):
    xb = x_ref[...]
    h = jnp.dot(xb, gW1_ref[...], preferred_element_type=jnp.float32)
    h = h + gb1_ref[...]
    h = h * jax.nn.sigmoid(h)
    logits = jnp.dot(h, gW2_ref[...], preferred_element_type=jnp.float32)
    logits = logits + gb2_ref[...]
    m = jnp.max(logits, axis=1, keepdims=True)
    ex = jnp.exp(logits - m)
    gw = ex / jnp.sum(ex, axis=1, keepdims=True)

    idx8 = lax.broadcasted_iota(jnp.int32, gw.shape, 1)
    m1 = jnp.max(gw, axis=1, keepdims=True)
    i1 = jnp.min(jnp.where(gw == m1, idx8, E), axis=1, keepdims=True)
    sel1 = idx8 == i1
    g2 = jnp.where(sel1, -1.0, gw)
    m2 = jnp.max(g2, axis=1, keepdims=True)
    i2 = jnp.min(jnp.where(g2 == m2, idx8, E), axis=1, keepdims=True)
    sel2 = idx8 == i2
    mask = (sel1 & (m1 > THRESH)) | (sel2 & (m2 > THRESH))

    gw_ref[...] = gw
    w_ref[...] = gw * mask.astype(jnp.float32)


def _route_body(w_ref, blend_ref, dest_ref, pad_ref, bem_ref, act_ref,
                pidx_ref, coef_ref):
    w = w_ref[...]
    mask = w > 0.0
    mi = mask.astype(jnp.int32)

    c = mi
    k = 1
    while k < B:
        c = c + jnp.concatenate(
            [jnp.zeros((k, E), jnp.int32), c[: B - k]], axis=0)
        k *= 2
    pos = c - mi
    counts = c[B - 1:B, :]
    padded = ((counts + (BLK_M - 1)) >> 9) << 9

    ii = lax.broadcasted_iota(jnp.int32, (E, E), 0)
    jj = lax.broadcasted_iota(jnp.int32, (E, E), 1)
    tri = (ii < jj).astype(jnp.float32)
    offs_f = jnp.dot(padded.astype(jnp.float32), tri,
                     preferred_element_type=jnp.float32)
    offs = offs_f.astype(jnp.int32)

    dest = jnp.where(mask, offs + pos, R)
    dest_ref[...] = dest

    jp = lax.broadcasted_iota(jnp.int32, (BLK_M, E), 0)
    pad_ref[...] = jnp.where(jp < padded - counts, offs + counts + jp, R)

    nblk = padded >> 9
    blk_off = offs >> 9
    ib = lax.broadcasted_iota(jnp.int32, (32, E), 0)
    bo = jnp.broadcast_to(blk_off, (32, E))
    nb = jnp.broadcast_to(nblk, (32, E))
    ind = (ib >= bo) & (ib < bo + nb)
    eidx = lax.broadcasted_iota(jnp.int32, (32, E), 1)
    bev = jnp.sum(jnp.where(ind, eidx, 0), axis=1, keepdims=True)
    actv = jnp.sum(ind.astype(jnp.int32), axis=1, keepdims=True)
    te = jnp.max(jnp.where(counts > 0,
                           lax.broadcasted_iota(jnp.int32, (1, E), 1), -1))
    te = jnp.maximum(te, 0)
    bem_ref[...] = jnp.where(actv > 0, bev, te)
    act_ref[...] = actv

    alpha = jax.nn.sigmoid(blend_ref[0, 0])
    e_idx = lax.broadcasted_iota(jnp.int32, (B, E), 1)
    e0 = jnp.min(jnp.where(mask, e_idx, 100), axis=1, keepdims=True)
    e1 = jnp.max(jnp.where(mask, e_idx, -1), axis=1, keepdims=True)
    nm = jnp.sum(mi, axis=1, keepdims=True)
    sel0 = (e_idx == e0) & mask
    sel1 = (e_idx == e1) & mask & (nm == 2)
    w0 = jnp.sum(jnp.where(sel0, w, 0.0), axis=1, keepdims=True)
    w1 = jnp.sum(jnp.where(sel1, w, 0.0), axis=1, keepdims=True)
    p0 = jnp.sum(jnp.where(sel0, dest, 0), axis=1, keepdims=True)
    p1 = jnp.sum(jnp.where(sel1, dest, 0), axis=1, keepdims=True)
    fired = nm > 0
    tw = w0 + w1
    denom = jnp.where(fired, tw, 1.0)
    cx = jnp.where(fired, 1.0 - alpha, 1.0)
    c0 = (alpha * w0) / denom
    c1 = (alpha * w1) / denom
    p0 = jnp.where(fired, p0, ZROW)
    p1 = jnp.where(nm == 2, p1, ZROW)
    zi = jnp.zeros((B, E - 2), jnp.int32)
    zf = jnp.zeros((B, E - 3), jnp.float32)
    pidx_ref[...] = jnp.concatenate([p0, p1, zi], axis=1)
    coef_ref[...] = jnp.concatenate([c0, c1, cx, zf], axis=1)


EPT = (B * E) // NW
PPT = (BLK_M * E) // NW


def _scat_body(dest_hbm, pad_hbm, ids_hbm, dch_v, tok_v, pch_v, ptok_v,
               sem1, sem2):
    wid = lax.axis_index("s") * NC + lax.axis_index("c")
    base = wid * EPT
    lane = lax.iota(jnp.int32, 16)
    pltpu.sync_copy(dest_hbm.at[pl.ds(base, EPT)], dch_v)
    pltpu.sync_copy(pad_hbm.at[pl.ds(wid * PPT, PPT)], pch_v)

    def fill(i, _):
        tok_v[pl.ds(i * 16, 16)] = lax.shift_right_logical(
            base + i * 16 + lane, 3)
        return 0
    lax.fori_loop(0, EPT // 16, fill, 0)

    zi = jnp.zeros((16,), jnp.int32)

    def fillz(i, _):
        ptok_v[pl.ds(i * 16, 16)] = zi
        return 0
    lax.fori_loop(0, PPT // 16, fillz, 0)

    h1 = pltpu.async_copy(tok_v, ids_hbm.at[dch_v], sem1)
    h2 = pltpu.async_copy(ptok_v, ids_hbm.at[pch_v], sem2)
    h1.wait()
    h2.wait()


def _gath_body(ids_hbm, x_hbm, xg_hbm, ids_v, gbuf0, gbuf1,
               gs0, gs1, ws0, ws1):
    wid = lax.axis_index("s") * NC + lax.axis_index("c")
    base = wid * RPW
    pltpu.sync_copy(ids_hbm.at[pl.ds(base, RPW)], ids_v)

    def clampstep(i, _):
        s = pl.ds(i * 16, 16)
        v = ids_v[s]
        ids_v[s] = jnp.minimum(jnp.maximum(v, 0), B - 1)
        return 0
    lax.fori_loop(0, RPW // 16, clampstep, 0)

    nch = RPW // GCH
    bufs = (gbuf0, gbuf1)
    gsems = (gs0, gs1)
    wsems = (ws0, ws1)

    def sg(g):
        idx = ids_v.at[pl.ds(g * GCH, GCH)]
        return pltpu.async_copy(x_hbm.at[idx], bufs[g % 2], gsems[g % 2])

    def sw(g):
        return pltpu.async_copy(
            bufs[g % 2], xg_hbm.at[pl.ds(base + g * GCH, GCH)], wsems[g % 2])

    gh, wh = {}, {}
    gh[0] = sg(0)
    for g in range(nch):
        if g + 1 < nch:
            if g - 1 in wh:
                wh[g - 1].wait()
            gh[g + 1] = sg(g + 1)
        gh[g].wait()
        wh[g] = sw(g)
    wh[nch - 2].wait()
    wh[nch - 1].wait()


def _gmm1_body(bem_ref, act_ref, xg_ref, Wp_ref, bp_ref, pg_ref):
    i = pl.program_id(0)

    @pl.when(act_ref[i] > 0)
    def _():
        p = jnp.dot(xg_ref[...], Wp_ref[0], preferred_element_type=jnp.float32)
        pg_ref[...] = p + bp_ref[0]


def _gmm2_body(bem_ref, act_ref, pg_ref, Wo_ref, yg_ref):
    i = pl.program_id(0)

    @pl.when(act_ref[i] > 0)
    def _():
        yg_ref[...] = jnp.dot(pg_ref[...], Wo_ref[0],
                              preferred_element_type=jnp.float32)

    @pl.when(act_ref[i] == 0)
    def _():
        yg_ref[...] = jnp.zeros_like(yg_ref)


def _comb_body(x_hbm, yg_hbm, p0_hbm, p1_hbm, c0_hbm, c1_hbm, cx_hbm, out_hbm,
               p0v, p1v, c0v, c1v, cxv,
               xbuf0, xbuf1, y0buf0, y0buf1, y1buf0, y1buf1, obuf,
               xs0, xs1, s00, s01, s10, s11, osem):
    wid = lax.axis_index("s") * NC + lax.axis_index("c")
    tbase = wid * TPW
    pltpu.sync_copy(p0_hbm.at[pl.ds(tbase, TPW)], p0v)
    pltpu.sync_copy(p1_hbm.at[pl.ds(tbase, TPW)], p1v)
    pltpu.sync_copy(c0_hbm.at[pl.ds(tbase, TPW)], c0v)
    pltpu.sync_copy(c1_hbm.at[pl.ds(tbase, TPW)], c1v)
    pltpu.sync_copy(cx_hbm.at[pl.ds(tbase, TPW)], cxv)

    nch = TPW // CCH
    xbufs, xsems = (xbuf0, xbuf1), (xs0, xs1)
    y0bufs, s0sems = (y0buf0, y0buf1), (s00, s01)
    y1bufs, s1sems = (y1buf0, y1buf1), (s10, s11)

    def start_in(c):
        b = c % 2
        row0 = tbase + c * CCH
        hx = pltpu.async_copy(x_hbm.at[pl.ds(row0, CCH)], xbufs[b], xsems[b])
        h0 = pltpu.async_copy(yg_hbm.at[p0v.at[pl.ds(c * CCH, CCH)]],
                              y0bufs[b], s0sems[b])
        h1 = pltpu.async_copy(yg_hbm.at[p1v.at[pl.ds(c * CCH, CCH)]],
                              y1bufs[b], s1sems[b])
        return (hx, h0, h1)

    ih, oh = {}, {}
    ih[0] = start_in(0)
    for c in range(nch):
        b = c % 2
        if c + 1 < nch:
            ih[c + 1] = start_in(c + 1)
        for h in ih[c]:
            h.wait()
        if c - 1 in oh:
            oh[c - 1].wait()
        xbuf, y0buf, y1buf = xbufs[b], y0bufs[b], y1bufs[b]
        for t in range(CCH):
            tvec = jnp.full((16,), c * CCH + t, jnp.int32)
            cxs = plsc.load_gather(cxv, [tvec])
            c0s = plsc.load_gather(c0v, [tvec])
            c1s = plsc.load_gather(c1v, [tvec])

            def jloop(j, _, t=t, cxs=cxs, c0s=c0s, c1s=c1s, xbuf=xbuf,
                      y0buf=y0buf, y1buf=y1buf, obuf=obuf):
                s = pl.ds(j * 16, 16)
                obuf[t, s] = (cxs * xbuf[t, s] + c0s * y0buf[t, s]
                              + c1s * y1buf[t, s])
                return 0
            lax.fori_loop(0, H // 16, jloop, 0)
        oh[c] = pltpu.async_copy(
            obuf, out_hbm.at[pl.ds(tbase + c * CCH, CCH)], osem)
    oh[nch - 1].wait()


@functools.lru_cache(maxsize=1)
def _sc_kernels():
    mesh = plsc.VectorSubcoreMesh(core_axis_name="c", subcore_axis_name="s")
    nlp = pltpu.CompilerParams(needs_layout_passes=False)
    scat = pl.kernel(
        _scat_body,
        out_type=jax.ShapeDtypeStruct((R + 8,), jnp.int32),
        mesh=mesh,
        scratch_types=[
            pltpu.VMEM((EPT,), jnp.int32),
            pltpu.VMEM((EPT,), jnp.int32),
            pltpu.VMEM((PPT,), jnp.int32),
            pltpu.VMEM((PPT,), jnp.int32),
            pltpu.SemaphoreType.DMA,
            pltpu.SemaphoreType.DMA,
        ],
        compiler_params=nlp,
    )
    gath = pl.kernel(
        _gath_body,
        out_type=jax.ShapeDtypeStruct((R, H), jnp.float32),
        mesh=mesh,
        scratch_types=[
            pltpu.VMEM((RPW,), jnp.int32),
            pltpu.VMEM((GCH, H), jnp.float32),
            pltpu.VMEM((GCH, H), jnp.float32),
            pltpu.SemaphoreType.DMA,
            pltpu.SemaphoreType.DMA,
            pltpu.SemaphoreType.DMA,
            pltpu.SemaphoreType.DMA,
        ],
        compiler_params=nlp,
    )
    comb = pl.kernel(
        _comb_body,
        out_type=jax.ShapeDtypeStruct((B, H), jnp.float32),
        mesh=mesh,
        scratch_types=(
            [
                pltpu.VMEM((TPW,), jnp.int32),
                pltpu.VMEM((TPW,), jnp.int32),
                pltpu.VMEM((TPW,), jnp.float32),
                pltpu.VMEM((TPW,), jnp.float32),
                pltpu.VMEM((TPW,), jnp.float32),
            ]
            + [pltpu.VMEM((CCH, H), jnp.float32)] * 7
            + [pltpu.SemaphoreType.DMA] * 7
        ),
        compiler_params=nlp,
    )
    return scat, gath, comb


def _gate_call(x, gW1, gb1, gW2, gb2):
    nb = B // BLK_B
    return pl.pallas_call(
        _gate_body,
        grid=(nb,),
        in_specs=[
            pl.BlockSpec((BLK_B, H), lambda i: (i, 0)),
            pl.BlockSpec((H, H // 2), lambda i: (0, 0)),
            pl.BlockSpec((1, H // 2), lambda i: (0, 0)),
            pl.BlockSpec((H // 2, E), lambda i: (0, 0)),
            pl.BlockSpec((1, E), lambda i: (0, 0)),
        ],
        out_specs=[
            pl.BlockSpec((BLK_B, E), lambda i: (i, 0)),
            pl.BlockSpec((BLK_B, E), lambda i: (i, 0)),
        ],
        out_shape=[
            jax.ShapeDtypeStruct((B, E), jnp.float32),
            jax.ShapeDtypeStruct((B, E), jnp.float32),
        ],
        compiler_params=pltpu.CompilerParams(
            dimension_semantics=("parallel",),
        ),
    )(x, gW1, gb1.reshape(1, H // 2), gW2, gb2.reshape(1, E))


def _route_call(w, blend):
    return pl.pallas_call(
        _route_body,
        in_specs=[
            pl.BlockSpec((B, E), lambda: (0, 0)),
            pl.BlockSpec(memory_space=pltpu.SMEM),
        ],
        out_specs=[
            pl.BlockSpec((B, E), lambda: (0, 0)),
            pl.BlockSpec((BLK_M, E), lambda: (0, 0)),
            pl.BlockSpec((32, 1), lambda: (0, 0)),
            pl.BlockSpec((32, 1), lambda: (0, 0)),
            pl.BlockSpec((B, E), lambda: (0, 0)),
            pl.BlockSpec((B, E), lambda: (0, 0)),
        ],
        out_shape=[
            jax.ShapeDtypeStruct((B, E), jnp.int32),
            jax.ShapeDtypeStruct((BLK_M, E), jnp.int32),
            jax.ShapeDtypeStruct((32, 1), jnp.int32),
            jax.ShapeDtypeStruct((32, 1), jnp.int32),
            jax.ShapeDtypeStruct((B, E), jnp.int32),
            jax.ShapeDtypeStruct((B, E), jnp.float32),
        ],
    )(w, blend.reshape(1, 1))


def _gmm_call(bem, act, xg, Wp, bp, Wo):
    safe = lambda i, bem, act: (jnp.where(act[i] > 0, i, 0), 0)
    wsel = lambda i, bem, act: (bem[i], 0, 0)
    spec1 = pltpu.PrefetchScalarGridSpec(
        num_scalar_prefetch=2,
        grid=(NBLK,),
        in_specs=[
            pl.BlockSpec((BLK_M, H), safe),
            pl.BlockSpec((1, H, H), wsel),
            pl.BlockSpec((1, 1, H), wsel),
        ],
        out_specs=pl.BlockSpec((BLK_M, H), lambda i, bem, act: (i, 0)),
    )
    pg = pl.pallas_call(
        _gmm1_body,
        grid_spec=spec1,
        out_shape=jax.ShapeDtypeStruct((R, H), jnp.float32),
        compiler_params=pltpu.CompilerParams(
            dimension_semantics=("arbitrary",),
        ),
    )(bem, act, xg, Wp, bp)

    spec2 = pltpu.PrefetchScalarGridSpec(
        num_scalar_prefetch=2,
        grid=(NBLK_G,),
        in_specs=[
            pl.BlockSpec((BLK_M, H), safe),
            pl.BlockSpec((1, H, H), wsel),
        ],
        out_specs=pl.BlockSpec((BLK_M, H), lambda i, bem, act: (i, 0)),
    )
    return pl.pallas_call(
        _gmm2_body,
        grid_spec=spec2,
        out_shape=jax.ShapeDtypeStruct((R + BLK_M, H), jnp.float32),
        compiler_params=pltpu.CompilerParams(
            dimension_semantics=("arbitrary",),
        ),
    )(bem, act, pg, Wo)


def kernel(x, gW1, gb1, gW2, gb2, Wp, bp, Wo, blend):
    gw, w = _gate_call(x, gW1, gb1, gW2, gb2)
    dest, pad, bem, act, pidx, coef = _route_call(w, blend)

    scat, gath, comb = _sc_kernels()
    ids = scat(dest.reshape(B * E), pad.reshape(BLK_M * E))
    xg = gath(ids, x)

    yg = _gmm_call(bem.reshape(32), act.reshape(32), xg,
                   Wp, bp.reshape(E, 1, H), Wo)

    out = comb(x, yg, pidx[:, 0], pidx[:, 1],
               coef[:, 0], coef[:, 1], coef[:, 2])
    return out, gw


# speedup vs baseline: 6.1953x; 6.1953x over previous
import functools

import jax
import jax.numpy as jnp
from jax import lax
from jax.experimental import pallas as pl
from jax.experimental.pallas import tpu as pltpu
from jax.experimental.pallas import tpu_sc as plsc

B = 4096
H = 2048
E = 8
THRESH = 0.1

BLK_B = 512
BLK_M = 512
NBLK = 24
R = NBLK * BLK_M
NBLK_G = NBLK + 1
ZROW = R
SENT = 1 << 20

NC, NS = 2, 16
NW = NC * NS
RPW = R // NW
GCH = 24
DCH = 4096
TPW = B // NW
CCH = 8


def _gate_body(x_ref, gW1_ref, gb1_ref, gW2_ref, gb2_ref, gw_ref, w_ref):
    xb = x_ref[...]
    h = jnp.dot(xb, gW1_ref[...], preferred_element_type=jnp.float32)
    h = h + gb1_ref[...]
    h = h * jax.nn.sigmoid(h)
    logits = jnp.dot(h, gW2_ref[...], preferred_element_type=jnp.float32)
    logits = logits + gb2_ref[...]
    m = jnp.max(logits, axis=1, keepdims=True)
    ex = jnp.exp(logits - m)
    gw = ex / jnp.sum(ex, axis=1, keepdims=True)

    idx8 = lax.broadcasted_iota(jnp.int32, gw.shape, 1)
    m1 = jnp.max(gw, axis=1, keepdims=True)
    i1 = jnp.min(jnp.where(gw == m1, idx8, E), axis=1, keepdims=True)
    sel1 = idx8 == i1
    g2 = jnp.where(sel1, -1.0, gw)
    m2 = jnp.max(g2, axis=1, keepdims=True)
    i2 = jnp.min(jnp.where(g2 == m2, idx8, E), axis=1, keepdims=True)
    sel2 = idx8 == i2
    mask = (sel1 & (m1 > THRESH)) | (sel2 & (m2 > THRESH))

    gw_ref[...] = gw
    w_ref[...] = gw * mask.astype(jnp.float32)


def _route_body(w_ref, blend_ref, dest_ref, bem_ref, act_ref,
                pidx_ref, coef_ref):
    w = w_ref[...]
    mask = w > 0.0
    mi = mask.astype(jnp.int32)

    c = mi
    k = 1
    while k < B:
        c = c + jnp.concatenate(
            [jnp.zeros((k, E), jnp.int32), c[: B - k]], axis=0)
        k *= 2
    pos = c - mi
    counts = c[B - 1:B, :]
    padded = ((counts + (BLK_M - 1)) >> 9) << 9

    ii = lax.broadcasted_iota(jnp.int32, (E, E), 0)
    jj = lax.broadcasted_iota(jnp.int32, (E, E), 1)
    tri = (ii < jj).astype(jnp.float32)
    offs_f = jnp.dot(padded.astype(jnp.float32), tri,
                     preferred_element_type=jnp.float32)
    offs = offs_f.astype(jnp.int32)

    dest = jnp.where(mask, offs + pos, R)
    dest_ref[...] = dest

    nblk = padded >> 9
    blk_off = offs >> 9
    ib = lax.broadcasted_iota(jnp.int32, (32, E), 0)
    bo = jnp.broadcast_to(blk_off, (32, E))
    nb = jnp.broadcast_to(nblk, (32, E))
    ind = (ib >= bo) & (ib < bo + nb)
    eidx = lax.broadcasted_iota(jnp.int32, (32, E), 1)
    bev = jnp.sum(jnp.where(ind, eidx, 0), axis=1, keepdims=True)
    actv = jnp.sum(ind.astype(jnp.int32), axis=1, keepdims=True)
    te = jnp.max(jnp.where(counts > 0,
                           lax.broadcasted_iota(jnp.int32, (1, E), 1), -1))
    te = jnp.maximum(te, 0)
    bem_ref[...] = jnp.where(actv > 0, bev, te)
    act_ref[...] = actv

    alpha = jax.nn.sigmoid(blend_ref[0, 0])
    e_idx = lax.broadcasted_iota(jnp.int32, (B, E), 1)
    e0 = jnp.min(jnp.where(mask, e_idx, 100), axis=1, keepdims=True)
    e1 = jnp.max(jnp.where(mask, e_idx, -1), axis=1, keepdims=True)
    nm = jnp.sum(mi, axis=1, keepdims=True)
    sel0 = (e_idx == e0) & mask
    sel1 = (e_idx == e1) & mask & (nm == 2)
    w0 = jnp.sum(jnp.where(sel0, w, 0.0), axis=1, keepdims=True)
    w1 = jnp.sum(jnp.where(sel1, w, 0.0), axis=1, keepdims=True)
    p0 = jnp.sum(jnp.where(sel0, dest, 0), axis=1, keepdims=True)
    p1 = jnp.sum(jnp.where(sel1, dest, 0), axis=1, keepdims=True)
    fired = nm > 0
    tw = w0 + w1
    denom = jnp.where(fired, tw, 1.0)
    cx = jnp.where(fired, 1.0 - alpha, 1.0)
    c0 = (alpha * w0) / denom
    c1 = (alpha * w1) / denom
    p0 = jnp.where(fired, p0, ZROW)
    p1 = jnp.where(nm == 2, p1, ZROW)
    zi = jnp.zeros((B, E - 2), jnp.int32)
    zf = jnp.zeros((B, E - 3), jnp.float32)
    pidx_ref[...] = jnp.concatenate([p0, p1, zi], axis=1)
    coef_ref[...] = jnp.concatenate([c0, c1, cx, zf], axis=1)


EPT = 8192
IDS_SH = R + 256
ZP = IDS_SH // NS


def _scga_body(dest_hbm, x_hbm, xg_hbm,
               ids_sh, zbuf, dch_v, tok_v, ids_v, gbuf0, gbuf1,
               ssem, gs0, gs1, ws0, ws1):
    sid = lax.axis_index("s")
    lane = lax.iota(jnp.int32, 16)
    zi = jnp.zeros((16,), jnp.int32)

    def fillz(i, _):
        zbuf[pl.ds(i * 16, 16)] = zi
        return 0
    lax.fori_loop(0, ZP // 16, fillz, 0)

    pltpu.sync_copy(zbuf, ids_sh.at[pl.ds(sid * ZP, ZP)])
    plsc.subcore_barrier()

    @pl.when(sid == 0)
    def _():
        def chunkstep(q, _):
            qbase = q * EPT
            pltpu.sync_copy(dest_hbm.at[pl.ds(qbase, EPT)], dch_v)

            def fill(i, _):
                tok_v[pl.ds(i * 16, 16)] = lax.shift_right_logical(
                    qbase + i * 16 + lane, 3)
                return 0
            lax.fori_loop(0, EPT // 16, fill, 0)
            pltpu.sync_copy(tok_v, ids_sh.at[dch_v], add=True)
            return 0
        lax.fori_loop(0, (B * E) // EPT, chunkstep, 0)
    plsc.subcore_barrier()

    wid = sid * NC + lax.axis_index("c")
    base = wid * RPW
    pltpu.sync_copy(ids_sh.at[pl.ds(base, RPW)], ids_v)

    nch = RPW // GCH
    bufs = (gbuf0, gbuf1)
    gsems = (gs0, gs1)
    wsems = (ws0, ws1)

    def sg(g):
        idx = ids_v.at[pl.ds(g * GCH, GCH)]
        return pltpu.async_copy(x_hbm.at[idx], bufs[g % 2], gsems[g % 2])

    def sw(g):
        return pltpu.async_copy(
            bufs[g % 2], xg_hbm.at[pl.ds(base + g * GCH, GCH)], wsems[g % 2])

    gh, wh = {}, {}
    gh[0] = sg(0)
    for g in range(nch):
        if g + 1 < nch:
            if g - 1 in wh:
                wh[g - 1].wait()
            gh[g + 1] = sg(g + 1)
        gh[g].wait()
        wh[g] = sw(g)
    wh[nch - 2].wait()
    wh[nch - 1].wait()


def _gmm1_body(bem_ref, act_ref, xg_ref, Wp_ref, bp_ref, pg_ref):
    i = pl.program_id(0)

    @pl.when(act_ref[i] > 0)
    def _():
        p = jnp.dot(xg_ref[...], Wp_ref[0], preferred_element_type=jnp.float32)
        pg_ref[...] = p + bp_ref[0]


def _gmm2_body(bem_ref, act_ref, pg_ref, Wo_ref, yg_ref):
    i = pl.program_id(0)

    @pl.when(act_ref[i] > 0)
    def _():
        yg_ref[...] = jnp.dot(pg_ref[...], Wo_ref[0],
                              preferred_element_type=jnp.float32)

    @pl.when(act_ref[i] == 0)
    def _():
        yg_ref[...] = jnp.zeros_like(yg_ref)


def _comb_body(x_hbm, yg_hbm, p0_hbm, p1_hbm, c0_hbm, c1_hbm, cx_hbm, out_hbm,
               p0v, p1v, c0v, c1v, cxv,
               xbuf0, xbuf1, y0buf0, y0buf1, y1buf0, y1buf1, obuf,
               xs0, xs1, s00, s01, s10, s11, osem):
    wid = lax.axis_index("s") * NC + lax.axis_index("c")
    tbase = wid * TPW
    pltpu.sync_copy(p0_hbm.at[pl.ds(tbase, TPW)], p0v)
    pltpu.sync_copy(p1_hbm.at[pl.ds(tbase, TPW)], p1v)
    pltpu.sync_copy(c0_hbm.at[pl.ds(tbase, TPW)], c0v)
    pltpu.sync_copy(c1_hbm.at[pl.ds(tbase, TPW)], c1v)
    pltpu.sync_copy(cx_hbm.at[pl.ds(tbase, TPW)], cxv)

    nch = TPW // CCH
    xbufs, xsems = (xbuf0, xbuf1), (xs0, xs1)
    y0bufs, s0sems = (y0buf0, y0buf1), (s00, s01)
    y1bufs, s1sems = (y1buf0, y1buf1), (s10, s11)

    def start_in(c):
        b = c % 2
        row0 = tbase + c * CCH
        hx = pltpu.async_copy(x_hbm.at[pl.ds(row0, CCH)], xbufs[b], xsems[b])
        h0 = pltpu.async_copy(yg_hbm.at[p0v.at[pl.ds(c * CCH, CCH)]],
                              y0bufs[b], s0sems[b])
        h1 = pltpu.async_copy(yg_hbm.at[p1v.at[pl.ds(c * CCH, CCH)]],
                              y1bufs[b], s1sems[b])
        return (hx, h0, h1)

    ih, oh = {}, {}
    ih[0] = start_in(0)
    for c in range(nch):
        b = c % 2
        if c + 1 < nch:
            ih[c + 1] = start_in(c + 1)
        for h in ih[c]:
            h.wait()
        if c - 1 in oh:
            oh[c - 1].wait()
        xbuf, y0buf, y1buf = xbufs[b], y0bufs[b], y1bufs[b]
        for t in range(CCH):
            tvec = jnp.full((16,), c * CCH + t, jnp.int32)
            cxs = plsc.load_gather(cxv, [tvec])
            c0s = plsc.load_gather(c0v, [tvec])
            c1s = plsc.load_gather(c1v, [tvec])

            def jloop(j, _, t=t, cxs=cxs, c0s=c0s, c1s=c1s, xbuf=xbuf,
                      y0buf=y0buf, y1buf=y1buf, obuf=obuf):
                s = pl.ds(j * 16, 16)
                obuf[t, s] = (cxs * xbuf[t, s] + c0s * y0buf[t, s]
                              + c1s * y1buf[t, s])
                return 0
            lax.fori_loop(0, H // 16, jloop, 0)
        oh[c] = pltpu.async_copy(
            obuf, out_hbm.at[pl.ds(tbase + c * CCH, CCH)], osem)
    oh[nch - 1].wait()


@functools.lru_cache(maxsize=1)
def _sc_kernels():
    mesh = plsc.VectorSubcoreMesh(core_axis_name="c", subcore_axis_name="s")
    nlp = pltpu.CompilerParams(needs_layout_passes=False)
    scga = pl.kernel(
        _scga_body,
        out_type=jax.ShapeDtypeStruct((R, H), jnp.float32),
        mesh=mesh,
        scratch_types=[
            pltpu.VMEM_SHARED((IDS_SH,), jnp.int32),
            pltpu.VMEM((ZP,), jnp.int32),
            pltpu.VMEM((EPT,), jnp.int32),
            pltpu.VMEM((EPT,), jnp.int32),
            pltpu.VMEM((RPW,), jnp.int32),
            pltpu.VMEM((GCH, H), jnp.float32),
            pltpu.VMEM((GCH, H), jnp.float32),
            pltpu.SemaphoreType.DMA,
            pltpu.SemaphoreType.DMA,
            pltpu.SemaphoreType.DMA,
            pltpu.SemaphoreType.DMA,
            pltpu.SemaphoreType.DMA,
        ],
        compiler_params=nlp,
    )
    comb = pl.kernel(
        _comb_body,
        out_type=jax.ShapeDtypeStruct((B, H), jnp.float32),
        mesh=mesh,
        scratch_types=(
            [
                pltpu.VMEM((TPW,), jnp.int32),
                pltpu.VMEM((TPW,), jnp.int32),
                pltpu.VMEM((TPW,), jnp.float32),
                pltpu.VMEM((TPW,), jnp.float32),
                pltpu.VMEM((TPW,), jnp.float32),
            ]
            + [pltpu.VMEM((CCH, H), jnp.float32)] * 7
            + [pltpu.SemaphoreType.DMA] * 7
        ),
        compiler_params=nlp,
    )
    return scga, comb


def _gate_call(x, gW1, gb1, gW2, gb2):
    nb = B // BLK_B
    return pl.pallas_call(
        _gate_body,
        grid=(nb,),
        in_specs=[
            pl.BlockSpec((BLK_B, H), lambda i: (i, 0)),
            pl.BlockSpec((H, H // 2), lambda i: (0, 0)),
            pl.BlockSpec((1, H // 2), lambda i: (0, 0)),
            pl.BlockSpec((H // 2, E), lambda i: (0, 0)),
            pl.BlockSpec((1, E), lambda i: (0, 0)),
        ],
        out_specs=[
            pl.BlockSpec((BLK_B, E), lambda i: (i, 0)),
            pl.BlockSpec((BLK_B, E), lambda i: (i, 0)),
        ],
        out_shape=[
            jax.ShapeDtypeStruct((B, E), jnp.float32),
            jax.ShapeDtypeStruct((B, E), jnp.float32),
        ],
        compiler_params=pltpu.CompilerParams(
            dimension_semantics=("parallel",),
        ),
    )(x, gW1, gb1.reshape(1, H // 2), gW2, gb2.reshape(1, E))


def _route_call(w, blend):
    return pl.pallas_call(
        _route_body,
        in_specs=[
            pl.BlockSpec((B, E), lambda: (0, 0)),
            pl.BlockSpec(memory_space=pltpu.SMEM),
        ],
        out_specs=[
            pl.BlockSpec((B, E), lambda: (0, 0)),
            pl.BlockSpec((32, 1), lambda: (0, 0)),
            pl.BlockSpec((32, 1), lambda: (0, 0)),
            pl.BlockSpec((B, E), lambda: (0, 0)),
            pl.BlockSpec((B, E), lambda: (0, 0)),
        ],
        out_shape=[
            jax.ShapeDtypeStruct((B, E), jnp.int32),
            jax.ShapeDtypeStruct((32, 1), jnp.int32),
            jax.ShapeDtypeStruct((32, 1), jnp.int32),
            jax.ShapeDtypeStruct((B, E), jnp.int32),
            jax.ShapeDtypeStruct((B, E), jnp.float32),
        ],
    )(w, blend.reshape(1, 1))


def _gmm_call(bem, act, xg, Wp, bp, Wo):
    safe = lambda i, bem, act: (jnp.where(act[i] > 0, i, 0), 0)
    wsel = lambda i, bem, act: (bem[i], 0, 0)
    spec1 = pltpu.PrefetchScalarGridSpec(
        num_scalar_prefetch=2,
        grid=(NBLK,),
        in_specs=[
            pl.BlockSpec((BLK_M, H), safe),
            pl.BlockSpec((1, H, H), wsel),
            pl.BlockSpec((1, 1, H), wsel),
        ],
        out_specs=pl.BlockSpec((BLK_M, H), lambda i, bem, act: (i, 0)),
    )
    pg = pl.pallas_call(
        _gmm1_body,
        grid_spec=spec1,
        out_shape=jax.ShapeDtypeStruct((R, H), jnp.float32),
        compiler_params=pltpu.CompilerParams(
            dimension_semantics=("arbitrary",),
        ),
    )(bem, act, xg, Wp, bp)

    spec2 = pltpu.PrefetchScalarGridSpec(
        num_scalar_prefetch=2,
        grid=(NBLK_G,),
        in_specs=[
            pl.BlockSpec((BLK_M, H), safe),
            pl.BlockSpec((1, H, H), wsel),
        ],
        out_specs=pl.BlockSpec((BLK_M, H), lambda i, bem, act: (i, 0)),
    )
    return pl.pallas_call(
        _gmm2_body,
        grid_spec=spec2,
        out_shape=jax.ShapeDtypeStruct((R + BLK_M, H), jnp.float32),
        compiler_params=pltpu.CompilerParams(
            dimension_semantics=("arbitrary",),
        ),
    )(bem, act, pg, Wo)


def kernel(x, gW1, gb1, gW2, gb2, Wp, bp, Wo, blend):
    gw, w = _gate_call(x, gW1, gb1, gW2, gb2)
    dest, bem, act, pidx, coef = _route_call(w, blend)

    scga, comb = _sc_kernels()
    xg = scga(dest.reshape(B * E), x)

    yg = _gmm_call(bem.reshape(32), act.reshape(32), xg,
                   Wp, bp.reshape(E, 1, H), Wo)

    out = comb(x, yg, pidx[:, 0], pidx[:, 1],
               coef[:, 0], coef[:, 1], coef[:, 2])
    return out, gw
